# Initial kernel scaffold; baseline (speedup 1.0000x reference)
import jax
import jax.numpy as jnp
from jax.experimental import pallas as pl


def kernel(x, edge_index, edge_attr, batch, W0, b0, W_rel, W_root, b_conv, W_ih, W_hh, b_ih, b_hh, W1, b1, W2, b2):
    raise NotImplementedError("write your pallas kernel here")


# speedup vs baseline: 9.0760x; 9.0760x over previous
import functools

import jax
import jax.numpy as jnp
from jax import lax
from jax.experimental import pallas as pl
from jax.experimental.pallas import tpu as pltpu
from jax.experimental.pallas import tpu_sc as plsc

N = 10000
E = 320000
D_IN = 128
D_H = 64
R = 5
B = 64
OUT = 12
PROP_STEPS = 6
S2S_STEPS = 6

HALF = 32
NSEG = N * R
TRASH = NSEG
CH = 128
NC = 2
NS = 16
NW = NC * NS
E_PAD = 323584
EPT = E_PAD // NS
ZROWS = 3200
A_ROWS = NS * ZROWS
OPT = 3128
OUT_HALF = NS * OPT


def _make_sc_kernel(do_gather):
    mesh = plsc.VectorSubcoreMesh(core_axis_name="c", subcore_axis_name="s")
    scratch = [
        pltpu.VMEM((CH,), jnp.int32),
        pltpu.VMEM((CH,), jnp.int32),
        pltpu.VMEM((CH, HALF), jnp.float32),
        pltpu.VMEM_SHARED((A_ROWS, HALF), jnp.float32),
        pltpu.SemaphoreType.DMA,
    ]

    def body(hsplit_hbm, src_hbm, seg_hbm, zeros_hbm, ones_hbm, out_hbm,
             srcv, segv, rows, acc, sem):
        c = lax.axis_index("c")
        s = lax.axis_index("s")
        pltpu.sync_copy(zeros_hbm, acc.at[pl.ds(s * ZROWS, ZROWS)])
        if not do_gather:
            pltpu.sync_copy(ones_hbm, rows)
        plsc.subcore_barrier()
        base = s * EPT

        def chunk(i, carry):
            off = base + i * CH
            pltpu.sync_copy(seg_hbm.at[pl.ds(off, CH)], segv)
            if do_gather:
                pltpu.sync_copy(src_hbm.at[pl.ds(off, CH)], srcv)
                for j in range(CH // 16):
                    sl = pl.ds(j * 16, 16)
                    srcv[sl] = srcv[sl] + c * N
                pltpu.async_copy(hsplit_hbm.at[srcv], rows, sem).wait()
            pltpu.sync_copy(rows, acc.at[segv], add=True)
            return carry

        lax.fori_loop(0, EPT // CH, chunk, 0)
        plsc.subcore_barrier()
        pltpu.sync_copy(
            acc.at[pl.ds(s * OPT, OPT)],
            out_hbm.at[pl.ds(c * OUT_HALF + s * OPT, OPT)])

    return functools.partial(
        pl.kernel, mesh=mesh,
        out_type=jax.ShapeDtypeStruct((NC * OUT_HALF, HALF), jnp.float32),
        scratch_types=scratch,
        compiler_params=pltpu.CompilerParams(use_tc_tiling_on_sc=False),
    )(body)


@functools.lru_cache(maxsize=None)
def _sc_kernel(do_gather):
    return _make_sc_kernel(do_gather)


def _lin0_body(x_ref, w0_ref, b0_ref, o_ref):
    h = lax.dot_general(x_ref[...], w0_ref[...], (((1,), (1,)), ((), ())),
                        preferred_element_type=jnp.float32)
    o_ref[...] = jnp.maximum(h + b0_ref[...], 0.0)


def _lin0(x, w0, b0):
    return pl.pallas_call(
        _lin0_body,
        out_shape=jax.ShapeDtypeStruct((N, D_H), jnp.float32),
    )(x, w0, b0)


def _step_body(h_ref, a_ref, cnt_ref, wrel_ref, wroot_ref, bconv_ref, o_ref):
    out = lax.dot_general(h_ref[...], wroot_ref[...], (((1,), (0,)), ((), ())),
                          preferred_element_type=jnp.float32) + bconv_ref[...]
    invc = 1.0 / jnp.maximum(cnt_ref[...], 1.0)
    for r in range(R):
        m = a_ref[r * N:(r + 1) * N, :] * invc[:, r:r + 1]
        out = out + lax.dot_general(m, wrel_ref[r], (((1,), (0,)), ((), ())),
                                    preferred_element_type=jnp.float32)
    o_ref[...] = jnp.maximum(out, 0.0)


def _step(h, a, cnt, wrel, wroot, bconv):
    return pl.pallas_call(
        _step_body,
        out_shape=jax.ShapeDtypeStruct((N, D_H), jnp.float32),
        compiler_params=pltpu.CompilerParams(vmem_limit_bytes=100 * 2**20),
    )(h, a, cnt, wrel, wroot, bconv)


def _s2s_body(h_ref, batch_ref, wih_ref, whh_ref, bih_ref, bhh_ref,
              w1_ref, b1_ref, w2_ref, b2_ref, o_ref):
    h = h_ref[...]
    p = (batch_ref[...] == lax.broadcasted_iota(jnp.int32, (N, B), 1)
         ).astype(jnp.float32)
    q_star = jnp.zeros((B, 2 * D_H), jnp.float32)
    hh = jnp.zeros((B, D_H), jnp.float32)
    cc = jnp.zeros((B, D_H), jnp.float32)
    for _ in range(S2S_STEPS):
        gates = (lax.dot_general(q_star, wih_ref[...], (((1,), (1,)), ((), ())),
                                 preferred_element_type=jnp.float32)
                 + bih_ref[...]
                 + lax.dot_general(hh, whh_ref[...], (((1,), (1,)), ((), ())),
                                   preferred_element_type=jnp.float32)
                 + bhh_ref[...])
        gi = jax.nn.sigmoid(gates[:, 0:D_H])
        gf = jax.nn.sigmoid(gates[:, D_H:2 * D_H])
        gg = jnp.tanh(gates[:, 2 * D_H:3 * D_H])
        go = jax.nn.sigmoid(gates[:, 3 * D_H:4 * D_H])
        cc = gf * cc + gi * gg
        hh = go * jnp.tanh(cc)
        q = hh
        qn = lax.dot_general(p, q, (((1,), (0,)), ((), ())),
                             preferred_element_type=jnp.float32)
        e = jnp.sum(h * qn, axis=1, keepdims=True)
        smat = p * e + (p - 1.0) * 1e30
        emax = jnp.max(smat, axis=0, keepdims=True)
        emax_n = lax.dot_general(p, emax, (((1,), (1,)), ((), ())),
                                 preferred_element_type=jnp.float32)
        ee = jnp.exp(e - emax_n)
        denom = lax.dot_general(p, ee, (((0,), (0,)), ((), ())),
                                preferred_element_type=jnp.float32)
        denom_n = lax.dot_general(p, denom, (((1,), (0,)), ((), ())),
                                  preferred_element_type=jnp.float32)
        a = ee / denom_n
        rvec = lax.dot_general(p, a * h, (((0,), (0,)), ((), ())),
                               preferred_element_type=jnp.float32)
        q_star = jnp.concatenate([q, rvec], axis=1)
    o1 = jnp.maximum(
        lax.dot_general(q_star, w1_ref[...], (((1,), (1,)), ((), ())),
                        preferred_element_type=jnp.float32) + b1_ref[...], 0.0)
    o_ref[...] = lax.dot_general(o1, w2_ref[...], (((1,), (1,)), ((), ())),
                                 preferred_element_type=jnp.float32) + b2_ref[...]


def _s2s(h, batch, wih, whh, bih, bhh, w1, b1, w2, b2):
    return pl.pallas_call(
        _s2s_body,
        out_shape=jax.ShapeDtypeStruct((B, OUT), jnp.float32),
    )(h, batch, wih, whh, bih, bhh, w1, b1, w2, b2)


def kernel(x, edge_index, edge_attr, batch, W0, b0, W_rel, W_root, b_conv,
           W_ih, W_hh, b_ih, b_hh, W1, b1, W2, b2):
    src = edge_index[0]
    dst = edge_index[1]
    seg = edge_attr * N + dst
    pad = E_PAD - E
    src_p = jnp.concatenate([src, jnp.zeros((pad,), jnp.int32)])
    seg_p = jnp.concatenate([seg, jnp.full((pad,), TRASH, jnp.int32)])
    zeros_blk = jnp.zeros((ZROWS, HALF), jnp.float32)
    ones_blk = jnp.ones((CH, HALF), jnp.float32)
    dummy_h = jnp.zeros((2 * N, HALF), jnp.float32)

    h = _lin0(x, W0, b0.reshape(1, -1))

    cnt_raw = _sc_kernel(False)(dummy_h, src_p, seg_p, zeros_blk, ones_blk)
    cnt = cnt_raw[:NSEG, 0].reshape(R, N).T

    for _ in range(PROP_STEPS):
        hsplit = h.reshape(N, 2, HALF).transpose(1, 0, 2).reshape(2 * N, HALF)
        a_raw = _sc_kernel(True)(hsplit, src_p, seg_p, zeros_blk, ones_blk)
        a = jnp.concatenate(
            [a_raw[:NSEG], a_raw[OUT_HALF:OUT_HALF + NSEG]], axis=1)
        h = _step(h, a, cnt, W_rel, W_root, b_conv.reshape(1, -1))

    return _s2s(h, batch.reshape(N, 1).astype(jnp.int32), W_ih, W_hh,
                b_ih.reshape(1, -1), b_hh.reshape(1, -1),
                W1, b1.reshape(1, -1), W2, b2.reshape(1, -1))
